# baseline (speedup 1.0000x reference)
import jax
import jax.numpy as jnp
from jax import lax
from jax.experimental import pallas as pl
from jax.experimental.pallas import tpu as pltpu

C = 256
_ACT_DT = jnp.bfloat16
_LEVELS = ((64, 64), (32, 32), (16, 16), (8, 8))
_NCHUNKS = (8, 2, 1, 1)


def _align8(n):
    return (n + 7) // 8 * 8


def _geom(H, W):
    Wp = W + 2
    N = H * Wp
    P = _align8(Wp + 1)
    M = _align8(P + N + Wp + 1)
    return Wp, N, P, M


def _conv_chunks(src, dst, w_slice, bias, H, W, nchunks, relu_mask, cout):
    Wp, N, P, _ = _geom(H, W)
    chunk = N // nchunks
    for i in range(nchunks):
        r0 = i * chunk
        acc = jnp.zeros((chunk, cout), jnp.float32)
        for ky in range(3):
            for kx in range(3):
                s = P + r0 + (ky - 1) * Wp + (kx - 1)
                xs = src[pl.ds(s, chunk), :]
                acc = acc + jnp.dot(xs, w_slice(ky * 3 + kx),
                                    preferred_element_type=jnp.float32)
        y = acc + bias
        if relu_mask:
            y = jnp.maximum(y, 0.0)
            col = (r0 + lax.broadcasted_iota(jnp.int32, (chunk, cout), 0)) % Wp
            y = jnp.where((col > 0) & (col < Wp - 1), y, 0.0)
            dst[pl.ds(P + r0, chunk), :] = y.astype(dst.dtype)
        else:
            dst[pl.ds(r0, chunk), :] = y


def _body(xh2, xh3, xh4, xh5, wmh, whh, bmh, bhh,
          oc2, ob2, oc3, ob3, oc4, ob4, oc5, ob5,
          x2, x3, x4, x5, wmv, whv, bmv, bhv, O, sems, *ab):
    xhs = (xh2, xh3, xh4, xh5)
    xvs = (x2, x3, x4, x5)
    outs = ((oc2, ob2), (oc3, ob3), (oc4, ob4), (oc5, ob5))
    copies = [(xh2, x2, 0), (xh3, x3, 1), (xh4, x4, 2), (xh5, x5, 3),
              (wmh, wmv, 4), (whh, whv, 5), (bmh, bmv, 6), (bhh, bhv, 7)]
    handles = [pltpu.make_async_copy(src, dst, sems.at[i])
               for src, dst, i in copies]
    for h in handles:
        h.start()
    for h in handles:
        h.wait()

    for l, (H, W) in enumerate(_LEVELS):
        Wp, N, P, M = _geom(H, W)
        A, B = ab[2 * l], ab[2 * l + 1]
        for buf in (A, B):
            buf[pl.ds(0, P), :] = jnp.zeros((P, C), buf.dtype)
            buf[pl.ds(P + N, M - P - N), :] = jnp.zeros((M - P - N, C),
                                                        buf.dtype)
        seq = (xvs[l], A, B, A, B)
        for layer in range(4):
            bias = bmv[layer]
            _conv_chunks(seq[layer], seq[layer + 1],
                         lambda t, layer=layer: wmv[layer, pl.ds(t * C, C), :],
                         bias, H, W, _NCHUNKS[l], True, C)
        _conv_chunks(B, O, lambda t: whv[pl.ds(t * C, C), :],
                     bhv[0:1, :], H, W, _NCHUNKS[l], False, 16)
        oc, ob = outs[l]
        yt = jnp.transpose(O[pl.ds(0, N), :], (1, 0))
        for h in range(H):
            row = yt[:, h * Wp + 1:h * Wp + 1 + W]
            oc[0, :, 0, h, :] = row[0:3, :]
            ob[0, :, :, h, :] = row[3:15, :].reshape(3, 4, W)


def kernel(p2, p3, p4, p5, w0, b0, w1, b1, w2, b2, w3, b3, wc, bc, wb, bb):
    xs = []
    for x, (H, W) in zip((p2, p3, p4, p5), _LEVELS):
        Wp, N, P, M = _geom(H, W)
        t = jnp.transpose(x[0].astype(_ACT_DT), (1, 2, 0))
        t = jnp.pad(t, ((0, 0), (1, 1), (0, 0)))
        t = t.reshape(N, C)
        t = jnp.pad(t, ((P, M - P - N), (0, 0)))
        xs.append(t)
    wm = jnp.stack([w.astype(_ACT_DT).transpose(2, 3, 1, 0).reshape(9 * C, C)
                    for w in (w0, w1, w2, w3)])
    whc = jnp.concatenate([wc, wb], axis=0).astype(_ACT_DT)
    wh = whc.transpose(2, 3, 1, 0).reshape(9 * C, 15)
    wh = jnp.pad(wh, ((0, 0), (0, 1)))
    bm = jnp.stack([b.reshape(1, C) for b in (b0, b1, b2, b3)])
    bh = jnp.pad(jnp.concatenate([bc, bb]), (0, 1)).reshape(1, 16)

    out_shape = []
    for H, W in _LEVELS:
        out_shape.append(jax.ShapeDtypeStruct((1, 3, 1, H, W), jnp.float32))
        out_shape.append(jax.ShapeDtypeStruct((1, 3, 4, H, W), jnp.float32))

    scratch = [pltpu.VMEM(x.shape, _ACT_DT) for x in xs]
    scratch += [pltpu.VMEM(wm.shape, _ACT_DT),
                pltpu.VMEM(wh.shape, _ACT_DT),
                pltpu.VMEM(bm.shape, jnp.float32),
                pltpu.VMEM(bh.shape, jnp.float32),
                pltpu.VMEM((_geom(64, 64)[1], 16), jnp.float32),
                pltpu.SemaphoreType.DMA((8,))]
    for H, W in _LEVELS:
        M = _geom(H, W)[3]
        scratch += [pltpu.VMEM((M, C), _ACT_DT),
                    pltpu.VMEM((M, C), _ACT_DT)]

    outs = pl.pallas_call(
        _body,
        out_shape=tuple(out_shape),
        in_specs=[pl.BlockSpec(memory_space=pl.ANY)] * 8,
        scratch_shapes=scratch,
    )(*xs, wm, wh, bm, bh)
    return tuple(outs)
